# baseline (speedup 1.0000x reference)
import functools

import jax
import jax.numpy as jnp
from jax import lax
from jax.experimental import pallas as pl
from jax.experimental.pallas import tpu as pltpu
from jax.experimental.pallas import tpu_sc as plsc

_NC = 2
_NS = 16
_NW = _NC * _NS
_L = 16

_ROWS = 100000
_D = 128
_N_GRAPH = 100
_RPG = _ROWS // _N_GRAPH
_G_SC = 44
_G_TC = _N_GRAPH - _G_SC

_TILE_ROWS = 125
_TILE_ELEMS = _TILE_ROWS * _D
_TILES_PER_GRAPH = _RPG // _TILE_ROWS
_N_TILES = _G_SC * _TILES_PER_GRAPH
_TPW = _N_TILES // _NW
_NBUF = 3
_UNROLL = 8

_mesh = plsc.VectorSubcoreMesh(
    core_axis_name="c", subcore_axis_name="s", num_cores=_NC, num_subcores=_NS
)


@functools.partial(
    pl.kernel,
    out_type=jax.ShapeDtypeStruct((_NW, _TPW, _L), jnp.float32),
    mesh=_mesh,
    scratch_types=(
        [pltpu.VMEM((_TILE_ELEMS,), jnp.float32) for _ in range(_NBUF)]
        + [pltpu.VMEM((_TILE_ELEMS,), jnp.float32) for _ in range(_NBUF)]
        + [pltpu.VMEM((_TPW, _L), jnp.float32)]
        + [pltpu.SemaphoreType.DMA for _ in range(_NBUF)]
    ),
)
def _sc_tile_sums(pred_hbm, tgt_hbm, out_hbm, *rest):
    pbufs = rest[:_NBUF]
    tbufs = rest[_NBUF:2 * _NBUF]
    osums = rest[2 * _NBUF]
    sems = rest[2 * _NBUF + 1:]
    w = lax.axis_index("s") * _NC + lax.axis_index("c")
    t0 = w * _TPW

    def start(j, b):
        off = (t0 + j) * _TILE_ELEMS
        pltpu.async_copy(pred_hbm.at[pl.ds(off, _TILE_ELEMS)], pbufs[b], sems[b])
        pltpu.async_copy(tgt_hbm.at[pl.ds(off, _TILE_ELEMS)], tbufs[b], sems[b])

    def wait(b):
        pltpu.make_async_copy(
            pred_hbm.at[pl.ds(0, _TILE_ELEMS)], pbufs[b], sems[b]).wait()
        pltpu.make_async_copy(
            tgt_hbm.at[pl.ds(0, _TILE_ELEMS)], tbufs[b], sems[b]).wait()

    def tile_sum(b):
        step = _L * _UNROLL

        def inner(i, carry):
            acc0, acc1 = carry
            base = i * step
            for u in range(_UNROLL):
                p = pbufs[b][pl.ds(base + u * _L, _L)]
                t = tbufs[b][pl.ds(base + u * _L, _L)]
                d = p - t
                if u % 2 == 0:
                    acc0 = acc0 + d * d
                else:
                    acc1 = acc1 + d * d
            return acc0, acc1

        z = jnp.zeros((_L,), jnp.float32)
        acc0, acc1 = lax.fori_loop(0, _TILE_ELEMS // step, inner, (z, z))
        return acc0 + acc1

    for b in range(_NBUF):
        start(b, b)

    def outer(k, _):
        for b in range(_NBUF):
            j = k * _NBUF + b

            @pl.when(j < _TPW)
            def _():
                wait(b)
                osums[j] = tile_sum(b)

                @pl.when(j + _NBUF < _TPW)
                def _():
                    start(j + _NBUF, b)

        return 0

    lax.fori_loop(0, (_TPW + _NBUF - 1) // _NBUF, outer, 0)
    pltpu.sync_copy(osums, out_hbm.at[w])


_GPB = 4


def _tc_body(p_ref, t_ref, o_ref):
    d = p_ref[...] - t_ref[...]
    o_ref[...] = jnp.sum(d * d, axis=1)[None]


_tc_graph_sums = pl.pallas_call(
    _tc_body,
    grid=(_G_TC // _GPB,),
    in_specs=[
        pl.BlockSpec((_GPB, _RPG, _D), lambda g: (g + _G_SC // _GPB, 0, 0)),
        pl.BlockSpec((_GPB, _RPG, _D), lambda g: (g + _G_SC // _GPB, 0, 0)),
    ],
    out_specs=pl.BlockSpec((1, _GPB, _D), lambda g: (g, 0, 0)),
    out_shape=jax.ShapeDtypeStruct((_G_TC // _GPB, _GPB, _D), jnp.float32),
)


def kernel(pred_nodes, target_nodes, n_node):
    sc_sums = _sc_tile_sums(pred_nodes.reshape(-1), target_nodes.reshape(-1))
    tc_sums = _tc_graph_sums(
        pred_nodes.reshape(_N_GRAPH, _RPG, _D),
        target_nodes.reshape(_N_GRAPH, _RPG, _D),
    )
    w = 0.5 / (
        jnp.maximum(n_node.astype(jnp.float32), 1.0) * (_N_GRAPH * _D)
    )
    r_sc = jnp.sum(
        sc_sums.reshape(_G_SC, _TILES_PER_GRAPH * _L) * w[:_G_SC, None]
    )
    r_tc = jnp.sum(tc_sums.reshape(_G_TC, _D) * w[_G_SC:, None])
    return r_sc + r_tc


# speedup vs baseline: 1.0072x; 1.0072x over previous
import functools

import jax
import jax.numpy as jnp
from jax import lax
from jax.experimental import pallas as pl
from jax.experimental.pallas import tpu as pltpu
from jax.experimental.pallas import tpu_sc as plsc

_NC = 2
_NS = 16
_NW = _NC * _NS
_L = 16

_ROWS = 100000
_D = 128
_N_GRAPH = 100
_RPG = _ROWS // _N_GRAPH
_G_SC = 44
_G_TC = _N_GRAPH - _G_SC

_TILE_ROWS = 125
_TILE_ELEMS = _TILE_ROWS * _D
_TILES_PER_GRAPH = _RPG // _TILE_ROWS
_N_TILES = _G_SC * _TILES_PER_GRAPH
_TPW = _N_TILES // _NW
_NBUF = 2
_UNROLL = 8

_mesh = plsc.VectorSubcoreMesh(
    core_axis_name="c", subcore_axis_name="s", num_cores=_NC, num_subcores=_NS
)


@functools.partial(
    pl.kernel,
    out_type=jax.ShapeDtypeStruct((_NW, _TPW, _L), jnp.float32),
    mesh=_mesh,
    scratch_types=(
        [pltpu.VMEM((_TILE_ELEMS,), jnp.float32) for _ in range(_NBUF)]
        + [pltpu.VMEM((_TILE_ELEMS,), jnp.float32) for _ in range(_NBUF)]
        + [pltpu.VMEM((_TPW, _L), jnp.float32)]
        + [pltpu.SemaphoreType.DMA for _ in range(_NBUF)]
    ),
)
def _sc_tile_sums(pred_hbm, tgt_hbm, out_hbm, *rest):
    pbufs = rest[:_NBUF]
    tbufs = rest[_NBUF:2 * _NBUF]
    osums = rest[2 * _NBUF]
    sems = rest[2 * _NBUF + 1:]
    w = lax.axis_index("s") * _NC + lax.axis_index("c")
    t0 = w * _TPW

    def start(j, b):
        off = (t0 + j) * _TILE_ELEMS
        pltpu.async_copy(pred_hbm.at[pl.ds(off, _TILE_ELEMS)], pbufs[b], sems[b])
        pltpu.async_copy(tgt_hbm.at[pl.ds(off, _TILE_ELEMS)], tbufs[b], sems[b])

    def wait(b):
        pltpu.make_async_copy(
            pred_hbm.at[pl.ds(0, _TILE_ELEMS)], pbufs[b], sems[b]).wait()
        pltpu.make_async_copy(
            tgt_hbm.at[pl.ds(0, _TILE_ELEMS)], tbufs[b], sems[b]).wait()

    def tile_sum(b):
        step = _L * _UNROLL

        def inner(i, carry):
            acc0, acc1 = carry
            base = i * step
            for u in range(_UNROLL):
                p = pbufs[b][pl.ds(base + u * _L, _L)]
                t = tbufs[b][pl.ds(base + u * _L, _L)]
                d = p - t
                if u % 2 == 0:
                    acc0 = acc0 + d * d
                else:
                    acc1 = acc1 + d * d
            return acc0, acc1

        z = jnp.zeros((_L,), jnp.float32)
        acc0, acc1 = lax.fori_loop(0, _TILE_ELEMS // step, inner, (z, z))
        return acc0 + acc1

    for b in range(_NBUF):
        start(b, b)

    def outer(k, _):
        for b in range(_NBUF):
            j = k * _NBUF + b

            @pl.when(j < _TPW)
            def _():
                wait(b)
                osums[j] = tile_sum(b)

                @pl.when(j + _NBUF < _TPW)
                def _():
                    start(j + _NBUF, b)

        return 0

    lax.fori_loop(0, (_TPW + _NBUF - 1) // _NBUF, outer, 0)
    pltpu.sync_copy(osums, out_hbm.at[w])


_GPB = 4


def _tc_body(p_ref, t_ref, o_ref):
    d = p_ref[...] - t_ref[...]
    o_ref[...] = jnp.sum(d * d, axis=1)[None]


_tc_graph_sums = pl.pallas_call(
    _tc_body,
    grid=(_G_TC // _GPB,),
    in_specs=[
        pl.BlockSpec((_GPB, _RPG, _D), lambda g: (g + _G_SC // _GPB, 0, 0)),
        pl.BlockSpec((_GPB, _RPG, _D), lambda g: (g + _G_SC // _GPB, 0, 0)),
    ],
    out_specs=pl.BlockSpec((1, _GPB, _D), lambda g: (g, 0, 0)),
    out_shape=jax.ShapeDtypeStruct((_G_TC // _GPB, _GPB, _D), jnp.float32),
)


def kernel(pred_nodes, target_nodes, n_node):
    sc_sums = _sc_tile_sums(pred_nodes.reshape(-1), target_nodes.reshape(-1))
    tc_sums = _tc_graph_sums(
        pred_nodes.reshape(_N_GRAPH, _RPG, _D),
        target_nodes.reshape(_N_GRAPH, _RPG, _D),
    )
    w = 0.5 / (
        jnp.maximum(n_node.astype(jnp.float32), 1.0) * (_N_GRAPH * _D)
    )
    r_sc = jnp.sum(
        sc_sums.reshape(_G_SC, _TILES_PER_GRAPH * _L) * w[:_G_SC, None]
    )
    r_tc = jnp.sum(tc_sums.reshape(_G_TC, _D) * w[_G_SC:, None])
    return r_sc + r_tc


# speedup vs baseline: 1.0235x; 1.0162x over previous
import functools

import jax
import jax.numpy as jnp
from jax import lax
from jax.experimental import pallas as pl
from jax.experimental.pallas import tpu as pltpu
from jax.experimental.pallas import tpu_sc as plsc

_NC = 2
_NS = 16
_NW = _NC * _NS
_L = 16

_ROWS = 100000
_D = 128
_N_GRAPH = 100
_RPG = _ROWS // _N_GRAPH
_G_SC = 44
_G_TC = _N_GRAPH - _G_SC

_TILE_ROWS = 125
_TILE_ELEMS = _TILE_ROWS * _D
_TILES_PER_GRAPH = _RPG // _TILE_ROWS
_N_TILES = _G_SC * _TILES_PER_GRAPH
_TPW = _N_TILES // _NW
_NBUF = 2
_UNROLL = 8

_mesh = plsc.VectorSubcoreMesh(
    core_axis_name="c", subcore_axis_name="s", num_cores=_NC, num_subcores=_NS
)


@functools.partial(
    pl.kernel,
    out_type=jax.ShapeDtypeStruct((_NW, _TPW, _L), jnp.float32),
    mesh=_mesh,
    scratch_types=[
        pltpu.VMEM((_NBUF, _TILE_ELEMS), jnp.float32),
        pltpu.VMEM((_NBUF, _TILE_ELEMS), jnp.float32),
        pltpu.VMEM((_TPW, _L), jnp.float32),
        pltpu.SemaphoreType.DMA,
        pltpu.SemaphoreType.DMA,
    ],
)
def _sc_tile_sums(pred_hbm, tgt_hbm, out_hbm, pbuf, tbuf, osums, sem0, sem1):
    sems = (sem0, sem1)
    w = lax.axis_index("s") * _NC + lax.axis_index("c")
    t0 = w * _TPW

    def start(j, b):
        off = (t0 + j) * _TILE_ELEMS
        pltpu.async_copy(pred_hbm.at[pl.ds(off, _TILE_ELEMS)], pbuf.at[b], sems[b])
        pltpu.async_copy(tgt_hbm.at[pl.ds(off, _TILE_ELEMS)], tbuf.at[b], sems[b])

    def wait(b):
        pltpu.make_async_copy(
            pred_hbm.at[pl.ds(0, _TILE_ELEMS)], pbuf.at[b], sems[b]).wait()
        pltpu.make_async_copy(
            tgt_hbm.at[pl.ds(0, _TILE_ELEMS)], tbuf.at[b], sems[b]).wait()

    def tile_sum(b):
        step = _L * _UNROLL

        def inner(i, carry):
            acc0, acc1 = carry
            base = i * step
            for u in range(_UNROLL):
                p = pbuf[b, pl.ds(base + u * _L, _L)]
                t = tbuf[b, pl.ds(base + u * _L, _L)]
                d = p - t
                if u % 2 == 0:
                    acc0 = acc0 + d * d
                else:
                    acc1 = acc1 + d * d
            return acc0, acc1

        z = jnp.zeros((_L,), jnp.float32)
        acc0, acc1 = lax.fori_loop(0, _TILE_ELEMS // step, inner, (z, z))
        return acc0 + acc1

    for b in range(_NBUF):
        start(b, b)

    def outer(k, _):
        for b in range(_NBUF):
            j = k * _NBUF + b

            @pl.when(j < _TPW)
            def _():
                wait(b)
                osums[j] = tile_sum(b)

                @pl.when(j + _NBUF < _TPW)
                def _():
                    start(j + _NBUF, b)

        return 0

    lax.fori_loop(0, (_TPW + _NBUF - 1) // _NBUF, outer, 0)
    pltpu.sync_copy(osums, out_hbm.at[w])


_GPB = 4


def _tc_body(p_ref, t_ref, o_ref):
    d = p_ref[...] - t_ref[...]
    o_ref[...] = jnp.sum(d * d, axis=1)[None]


_tc_graph_sums = pl.pallas_call(
    _tc_body,
    grid=(_G_TC // _GPB,),
    in_specs=[
        pl.BlockSpec((_GPB, _RPG, _D), lambda g: (g + _G_SC // _GPB, 0, 0)),
        pl.BlockSpec((_GPB, _RPG, _D), lambda g: (g + _G_SC // _GPB, 0, 0)),
    ],
    out_specs=pl.BlockSpec((1, _GPB, _D), lambda g: (g, 0, 0)),
    out_shape=jax.ShapeDtypeStruct((_G_TC // _GPB, _GPB, _D), jnp.float32),
)


def kernel(pred_nodes, target_nodes, n_node):
    sc_sums = _sc_tile_sums(pred_nodes.reshape(-1), target_nodes.reshape(-1))
    tc_sums = _tc_graph_sums(
        pred_nodes.reshape(_N_GRAPH, _RPG, _D),
        target_nodes.reshape(_N_GRAPH, _RPG, _D),
    )
    w = 0.5 / (
        jnp.maximum(n_node.astype(jnp.float32), 1.0) * (_N_GRAPH * _D)
    )
    r_sc = jnp.sum(
        sc_sums.reshape(_G_SC, _TILES_PER_GRAPH * _L) * w[:_G_SC, None]
    )
    r_tc = jnp.sum(tc_sums.reshape(_G_TC, _D) * w[_G_SC:, None])
    return r_sc + r_tc
